# baseline (speedup 1.0000x reference)
import jax
import jax.numpy as jnp
from jax.experimental import pallas as pl
from jax.experimental.pallas import tpu as pltpu

EPS = 1e-5
NEG_SLOPE = 0.01
LANE = 128


def _bn_train(x, gamma, beta):
    mu = jnp.mean(x, axis=0, keepdims=True)
    var = jnp.mean((x - mu) * (x - mu), axis=0, keepdims=True)
    return (x - mu) * jax.lax.rsqrt(var + EPS) * gamma + beta


def _leaky_relu(x):
    return jnp.where(x > 0, x, NEG_SLOPE * x)


def _make_pool_kernel(N, c_tile, HW, hw_tile, grid_k, inv_hw):
    small_hw = HW < LANE
    n_chunks = 1 if small_hw else hw_tile // LANE
    needs_mask = (not small_hw) and (HW % hw_tile != 0)
    last_k = grid_k - 1

    def plain_sum(block):
        s = block[:, :, 0:LANE]
        for i in range(1, n_chunks):
            s = s + block[:, :, i * LANE:(i + 1) * LANE]
        return s

    def tail_sum(block):
        base = last_k * hw_tile
        s = jnp.zeros((N, c_tile, LANE), jnp.float32)
        for i in range(n_chunks):
            lo = base + i * LANE
            if lo >= HW:
                break
            chunk = block[:, :, i * LANE:(i + 1) * LANE]
            if lo + LANE > HW:
                lane = jax.lax.broadcasted_iota(jnp.int32, (N, c_tile, LANE), 2)
                chunk = jnp.where(lane < (HW - lo), chunk, 0.0)
            s = s + chunk
        return s

    def _kernel_body(feats_ref, out_ref, acc_ref):
        k = pl.program_id(1)

        @pl.when(k == 0)
        def _():
            acc_ref[...] = jnp.zeros_like(acc_ref)

        block = feats_ref[...].astype(jnp.float32)

        if small_hw:
            acc_ref[...] += block
        elif not needs_mask:
            acc_ref[...] += plain_sum(block)
        else:
            @pl.when(k < last_k)
            def _():
                acc_ref[...] += plain_sum(block)

            @pl.when(k == last_k)
            def _():
                acc_ref[...] += tail_sum(block)

        @pl.when(k == last_k)
        def _():
            out_ref[...] = jnp.sum(acc_ref[...], axis=-1) * inv_hw

    return _kernel_body


def _make_pool_kernel_ms(n_streams, n_chunks, inv_hw):
    def _kernel_body(*refs):
        out_ref = refs[-1]
        s = None
        for r in refs[:-1]:
            x = r[...].astype(jnp.float32)
            for i in range(n_chunks):
                c = x[:, :, i * LANE:(i + 1) * LANE]
                s = c if s is None else s + c
        out_ref[...] = jnp.sum(s, axis=-1) * inv_hw
    return _kernel_body


def _pooled_means_ms(feats3, N, C, HW, c_tile, n_c, n_streams, itemsize):
    hw_s = HW // n_streams
    specs = [
        pl.BlockSpec((N, c_tile, hw_s), lambda ci, j=j: (0, ci, j))
        for j in range(n_streams)
    ]
    return pl.pallas_call(
        _make_pool_kernel_ms(n_streams, hw_s // LANE, 1.0 / float(HW)),
        out_shape=jax.ShapeDtypeStruct((N, C), jnp.float32),
        grid=(n_c,),
        in_specs=specs,
        out_specs=pl.BlockSpec((N, c_tile), lambda ci: (0, ci)),
        compiler_params=pltpu.CompilerParams(
            dimension_semantics=("parallel",),
            vmem_limit_bytes=96 << 20,
        ),
        cost_estimate=pl.CostEstimate(
            flops=int(N * C * HW),
            transcendentals=0,
            bytes_accessed=int(N * C * HW * itemsize + N * C * 4),
        ),
    )(*([feats3] * n_streams))


def _pooled_means(feats_nchw):
    N, C, H, W = feats_nchw.shape
    HW = H * W
    feats3 = feats_nchw.reshape(N, C, HW)
    itemsize = jnp.dtype(feats3.dtype).itemsize

    c_tile = 128 if C % 128 == 0 else C
    n_c = C // c_tile

    if HW >= LANE and HW % (LANE * 4) == 0 and n_c > 1:
        return _pooled_means_ms(feats3, N, C, HW, c_tile, n_c, 4, itemsize)

    if HW < LANE:
        hw_tile = HW
    elif HW % 1024 == 0 and N * c_tile * 1024 * itemsize <= (16 << 20):
        hw_tile = 1024
    elif HW % 512 == 0:
        hw_tile = 512
    elif HW % 256 == 0:
        hw_tile = 256
    else:
        hw_tile = LANE
    grid_k = pl.cdiv(HW, hw_tile)

    acc_lanes = LANE if HW >= LANE else HW

    pooled = pl.pallas_call(
        _make_pool_kernel(N, c_tile, HW, hw_tile, grid_k, 1.0 / float(HW)),
        out_shape=jax.ShapeDtypeStruct((N, C), jnp.float32),
        grid=(n_c, grid_k),
        in_specs=[pl.BlockSpec((N, c_tile, hw_tile), lambda ci, k: (0, ci, k))],
        out_specs=pl.BlockSpec((N, c_tile), lambda ci, k: (0, ci)),
        scratch_shapes=[pltpu.VMEM((N, c_tile, acc_lanes), jnp.float32)],
        compiler_params=pltpu.CompilerParams(
            dimension_semantics=("parallel", "arbitrary"),
            vmem_limit_bytes=96 << 20,
        ),
        cost_estimate=pl.CostEstimate(
            flops=int(N * C * HW),
            transcendentals=0,
            bytes_accessed=int(N * C * HW * itemsize + N * C * 4),
        ),
    )(feats3)
    return pooled


def _epilogue_kernel(feats_ref,
                     w1_ref, b1_ref, g1_ref, be1_ref,
                     w2_ref, b2_ref, g2_ref, be2_ref,
                     w3_ref, b3_ref, g3_ref, be3_ref,
                     w4_ref, b4_ref, wm_ref, bm_ref,
                     wd_ref, bd_ref, wc_ref, bc_ref,
                     set_ref, cls_ref, mix_ref):
    feats = feats_ref[...]

    h = jnp.dot(feats, w1_ref[...], preferred_element_type=jnp.float32) + b1_ref[...]
    h = _leaky_relu(_bn_train(h, g1_ref[...], be1_ref[...]))
    h = jnp.dot(h, w2_ref[...], preferred_element_type=jnp.float32) + b2_ref[...]
    h = _leaky_relu(_bn_train(h, g2_ref[...], be2_ref[...]))
    h = jnp.dot(h, w3_ref[...], preferred_element_type=jnp.float32) + b3_ref[...]
    h = _leaky_relu(_bn_train(h, g3_ref[...], be3_ref[...]))

    h4 = jnp.dot(h, w4_ref[...], preferred_element_type=jnp.float32) + b4_ref[...]
    mix = jax.nn.sigmoid(
        jnp.dot(h4, wm_ref[...], preferred_element_type=jnp.float32) + bm_ref[...])

    set_info = feats * mix
    class_info = feats - set_info
    set_ref[...] = (
        jnp.dot(set_info, wd_ref[...], preferred_element_type=jnp.float32) + bd_ref[...])
    cls_ref[...] = (
        jnp.dot(class_info, wc_ref[...], preferred_element_type=jnp.float32) + bc_ref[...])
    mix_ref[...] = mix


def kernel(feats, w1, b1, g1, be1, w2, b2, g2, be2, w3, b3, g3, be3,
           w4, b4, wm, bm, wd, bd, wc, bc):
    N, C, H, W = feats.shape
    K = wc.shape[1]

    pooled = _pooled_means(feats)

    set_preds, class_preds, mix_factor = pl.pallas_call(
        _epilogue_kernel,
        out_shape=(
            jax.ShapeDtypeStruct((N, 1), jnp.float32),
            jax.ShapeDtypeStruct((N, K), jnp.float32),
            jax.ShapeDtypeStruct((N, C), jnp.float32),
        ),
        compiler_params=pltpu.CompilerParams(vmem_limit_bytes=64 << 20),
    )(pooled,
      w1, b1, g1, be1,
      w2, b2, g2, be2,
      w3, b3, g3, be3,
      w4, b4, wm, bm, wd, bd, wc, bc)

    return set_preds, class_preds, mix_factor


# speedup vs baseline: 2.8666x; 2.8666x over previous
import jax
import jax.numpy as jnp
from jax.experimental import pallas as pl
from jax.experimental.pallas import tpu as pltpu

EPS = 1e-5
NEG_SLOPE = 0.01
LANE = 128


def _bn_train(x, gamma, beta):
    mu = jnp.mean(x, axis=0, keepdims=True)
    var = jnp.mean((x - mu) * (x - mu), axis=0, keepdims=True)
    return (x - mu) * jax.lax.rsqrt(var + EPS) * gamma + beta


def _leaky_relu(x):
    return jnp.where(x > 0, x, NEG_SLOPE * x)


def _make_pool_kernel(grid_k, inv_hw):
    last_k = grid_k - 1

    def _kernel_body(x_ref, out_ref):
        partial = jnp.sum(x_ref[...], axis=1)
        if grid_k == 1:
            out_ref[...] = partial * inv_hw
        else:
            k = pl.program_id(1)

            @pl.when(k == 0)
            def _():
                out_ref[...] = partial

            @pl.when(jnp.logical_and(k > 0, k < last_k))
            def _():
                out_ref[...] += partial

            @pl.when(k == last_k)
            def _():
                out_ref[...] = (out_ref[...] + partial) * inv_hw

    return _kernel_body


def _choose_hw_tile(HW, n_tile, C, itemsize, budget_bytes=4 << 20):
    best = None
    cap = max(1, budget_bytes // (n_tile * C * itemsize))
    for t in range(1, HW + 1):
        if HW % t == 0 and t % 8 == 0 and t <= cap:
            best = t
    if best is None:
        for t in range(1, HW + 1):
            if HW % t == 0 and t <= cap:
                best = t
    return best or HW


def _pooled_means(feats_nchw):
    N, C, H, W = feats_nchw.shape
    HW = H * W
    itemsize = jnp.dtype(feats_nchw.dtype).itemsize
    xt = jnp.transpose(feats_nchw, (0, 2, 3, 1)).reshape(N, HW, C)

    n_split = 2 if (N % 2 == 0 and (N // 2) % 8 == 0) else 1
    n_tile = N // n_split
    hw_tile = _choose_hw_tile(HW, n_tile, C, itemsize)
    grid_k = HW // hw_tile

    pooled = pl.pallas_call(
        _make_pool_kernel(grid_k, 1.0 / float(HW)),
        out_shape=jax.ShapeDtypeStruct((N, C), jnp.float32),
        grid=(n_split, grid_k),
        in_specs=[pl.BlockSpec((n_tile, hw_tile, C), lambda i, k: (i, k, 0))],
        out_specs=pl.BlockSpec((n_tile, C), lambda i, k: (i, 0)),
        compiler_params=pltpu.CompilerParams(
            dimension_semantics=("parallel", "arbitrary"),
            vmem_limit_bytes=96 << 20,
        ),
        cost_estimate=pl.CostEstimate(
            flops=int(N * C * HW),
            transcendentals=0,
            bytes_accessed=int(N * C * HW * itemsize + N * C * 4),
        ),
    )(xt)
    return pooled


def _epilogue_kernel(feats_ref,
                     w1_ref, b1_ref, g1_ref, be1_ref,
                     w2_ref, b2_ref, g2_ref, be2_ref,
                     w3_ref, b3_ref, g3_ref, be3_ref,
                     w4_ref, b4_ref, wm_ref, bm_ref,
                     wd_ref, bd_ref, wc_ref, bc_ref,
                     set_ref, cls_ref, mix_ref):
    feats = feats_ref[...]

    h = jnp.dot(feats, w1_ref[...], preferred_element_type=jnp.float32) + b1_ref[...]
    h = _leaky_relu(_bn_train(h, g1_ref[...], be1_ref[...]))
    h = jnp.dot(h, w2_ref[...], preferred_element_type=jnp.float32) + b2_ref[...]
    h = _leaky_relu(_bn_train(h, g2_ref[...], be2_ref[...]))
    h = jnp.dot(h, w3_ref[...], preferred_element_type=jnp.float32) + b3_ref[...]
    h = _leaky_relu(_bn_train(h, g3_ref[...], be3_ref[...]))

    h4 = jnp.dot(h, w4_ref[...], preferred_element_type=jnp.float32) + b4_ref[...]
    mix = jax.nn.sigmoid(
        jnp.dot(h4, wm_ref[...], preferred_element_type=jnp.float32) + bm_ref[...])

    set_info = feats * mix
    class_info = feats - set_info
    set_ref[...] = (
        jnp.dot(set_info, wd_ref[...], preferred_element_type=jnp.float32) + bd_ref[...])
    cls_ref[...] = (
        jnp.dot(class_info, wc_ref[...], preferred_element_type=jnp.float32) + bc_ref[...])
    mix_ref[...] = mix


def kernel(feats, w1, b1, g1, be1, w2, b2, g2, be2, w3, b3, g3, be3,
           w4, b4, wm, bm, wd, bd, wc, bc):
    N, C, H, W = feats.shape
    K = wc.shape[1]

    pooled = _pooled_means(feats)

    set_preds, class_preds, mix_factor = pl.pallas_call(
        _epilogue_kernel,
        out_shape=(
            jax.ShapeDtypeStruct((N, 1), jnp.float32),
            jax.ShapeDtypeStruct((N, K), jnp.float32),
            jax.ShapeDtypeStruct((N, C), jnp.float32),
        ),
        compiler_params=pltpu.CompilerParams(vmem_limit_bytes=64 << 20),
    )(pooled,
      w1, b1, g1, be1,
      w2, b2, g2, be2,
      w3, b3, g3, be3,
      w4, b4, wm, bm, wd, bd, wc, bc)

    return set_preds, class_preds, mix_factor
